# baseline (speedup 1.0000x reference)
import functools

import jax
import jax.numpy as jnp
from jax import lax
from jax.experimental import pallas as pl
from jax.experimental.pallas import tpu as pltpu
from jax.experimental.pallas import tpu_sc as plsc

N_NODE = 10000
N_EDGE = 160000
D = 128

NC, NS = 2, 16
NW = NC * NS
CHUNK = 128
N_CHUNKS = N_EDGE // CHUNK
CPW = (N_CHUNKS + NW - 1) // NW
CPW_LAST = N_CHUNKS - (NW - 1) * CPW
N_CHUNKS_PAD = NW * CPW
DEPTH = 2
GROUPS = CPW // DEPTH

SEG_CHUNK = 64
SEG_NCH = N_EDGE // SEG_CHUNK
SEG_CPW = 80
SEG_LAST = SEG_NCH - (NW - 1) * SEG_CPW
SEG_PAD = NW * SEG_CPW
SEG_DEPTH = 4
SEG_GROUPS = SEG_CPW // SEG_DEPTH
SEG_SUB = 64

NG_FULL = N_NODE // CHUNK
NG_TAIL = N_NODE - NG_FULL * CHUNK
NG_ITERS = (NG_FULL + NW - 1) // NW

H_SLC = 640
A_SLC = 640
A_SUB = CHUNK

_mesh = plsc.VectorSubcoreMesh(core_axis_name="c", subcore_axis_name="s")


def _worker_id():
  cid = lax.axis_index("c")
  sid = lax.axis_index("s")
  return cid, sid, sid * NC + cid


@functools.partial(
    pl.kernel,
    out_type=(
        jax.ShapeDtypeStruct((N_NODE, D), jnp.float32),
        jax.ShapeDtypeStruct((NC * N_NODE,), jnp.float32),
        jax.ShapeDtypeStruct((NC * N_NODE,), jnp.float32),
    ),
    mesh=_mesh,
    scratch_types=(
        tuple(pltpu.VMEM((CHUNK,), jnp.int32) for _ in range(NG_ITERS + 1)),
        pltpu.VMEM((CPW, CHUNK), jnp.int32),
        pltpu.VMEM((CPW, CHUNK), jnp.int32),
        tuple(pltpu.VMEM((CHUNK, D), jnp.float32) for _ in range(NG_ITERS + 1)),
        pltpu.VMEM((CHUNK,), jnp.float32),
        pltpu.VMEM((H_SLC,), jnp.float32),
        pltpu.VMEM_SHARED((N_NODE,), jnp.float32),
        pltpu.VMEM_SHARED((N_NODE,), jnp.float32),
        tuple(pltpu.SemaphoreType.DMA for _ in range(NG_ITERS + 1)),
    ),
)
def _sc_nodes_and_cp(emb_hbm, xpid_hbm, rdst_hbm, pdst_hbm, ones_hbm, zh_hbm,
                     xp_hbm, cp_hbm, cr_hbm,
                     idxs, ridx, didx, rows, ones_v, hbuf, cp_acc, cr_acc,
                     sems):
  cid, sid, wid = _worker_id()
  nch = jnp.where(wid == NW - 1, CPW_LAST, CPW)
  pltpu.sync_copy(ones_hbm, ones_v)
  hbase = jnp.minimum(sid * H_SLC, N_NODE - H_SLC)

  for c in range(NG_ITERS):
    g = c * NW + wid

    @pl.when(g < NG_FULL)
    def _(c=c, g=g):
      pltpu.sync_copy(xpid_hbm.at[pl.ds(g * CHUNK, CHUNK)], idxs[c])
      pltpu.make_async_copy(emb_hbm.at[idxs[c]], rows[c], sems[c]).start()

  tail = pl.ds(0, NG_TAIL)

  @pl.when(wid == 0)
  def _():
    pltpu.sync_copy(xpid_hbm.at[pl.ds(NG_FULL * CHUNK, NG_TAIL)],
                    idxs[NG_ITERS].at[tail])
    pltpu.make_async_copy(emb_hbm.at[idxs[NG_ITERS].at[tail]],
                          rows[NG_ITERS].at[tail], sems[NG_ITERS]).start()

  pltpu.sync_copy(rdst_hbm.at[pl.ds(wid * CPW, CPW), :], ridx)
  pltpu.sync_copy(pdst_hbm.at[pl.ds(wid * CPW, CPW), :], didx)

  pltpu.sync_copy(zh_hbm, hbuf)
  pltpu.sync_copy(hbuf, cp_acc.at[pl.ds(hbase, H_SLC)])
  pltpu.sync_copy(hbuf, cr_acc.at[pl.ds(hbase, H_SLC)])

  for c in range(NG_ITERS):
    g = c * NW + wid

    @pl.when(g < NG_FULL)
    def _(c=c, g=g):
      pltpu.make_async_copy(emb_hbm.at[idxs[c]], rows[c], sems[c]).wait()
      pltpu.sync_copy(rows[c], xp_hbm.at[pl.ds(g * CHUNK, CHUNK), :])

  @pl.when(wid == 0)
  def _():
    pltpu.make_async_copy(emb_hbm.at[idxs[NG_ITERS].at[tail]],
                          rows[NG_ITERS].at[tail], sems[NG_ITERS]).wait()
    pltpu.sync_copy(rows[NG_ITERS].at[tail],
                    xp_hbm.at[pl.ds(NG_FULL * CHUNK, NG_TAIL), :])

  plsc.subcore_barrier()

  def body(q, carry):
    descs = []
    for t in range(2):
      j = q * 2 + t
      d0 = pltpu.make_async_copy(ones_v, cp_acc.at[ridx.at[j]], sems[2 * t])
      d1 = pltpu.make_async_copy(ones_v, cr_acc.at[didx.at[j]],
                                 sems[2 * t + 1])
      descs.append((j, d0))
      descs.append((j, d1))

      @pl.when(j < nch)
      def _(d0=d0, d1=d1):
        d0.start(add=True)
        d1.start(add=True)

    for (j, d) in descs:
      @pl.when(j < nch)
      def _(d=d):
        d.wait()

    return carry

  lax.fori_loop(0, CPW // 2, body, 0)
  plsc.subcore_barrier()

  pltpu.sync_copy(cp_acc.at[pl.ds(hbase, H_SLC)], hbuf)
  pltpu.sync_copy(hbuf, cp_hbm.at[pl.ds(cid * N_NODE + hbase, H_SLC)])
  pltpu.sync_copy(cr_acc.at[pl.ds(hbase, H_SLC)], hbuf)
  pltpu.sync_copy(hbuf, cr_hbm.at[pl.ds(cid * N_NODE + hbase, H_SLC)])


def _make_seg_sum(with_hist):
  out_type = [jax.ShapeDtypeStruct((NC * N_NODE, D), jnp.float32)]
  if with_hist:
    out_type.append(jax.ShapeDtypeStruct((NC * N_NODE,), jnp.float32))

  @functools.partial(
      pl.kernel,
      out_type=tuple(out_type),
      mesh=_mesh,
      scratch_types=(
          pltpu.VMEM((SEG_CPW // 2, SEG_CHUNK), jnp.int32),
          pltpu.VMEM((SEG_CPW // 2, SEG_CHUNK), jnp.int32),
          tuple(pltpu.VMEM((SEG_CHUNK, D), jnp.float32)
                for _ in range(SEG_DEPTH)),
          pltpu.VMEM_SHARED((N_NODE, D), jnp.float32),
          tuple(pltpu.SemaphoreType.DMA for _ in range(SEG_DEPTH)),
      ),
  )
  def seg_sum(table_hbm, src_hbm, dst_hbm, ones_hbm, zrows_hbm, zh_hbm,
              *rest):
    (s_hbm, isrc, idst, rows, acc, sems) = rest
    cid, sid, wid = _worker_id()
    nch = jnp.where(wid == NW - 1, SEG_LAST, SEG_CPW)
    half = SEG_CPW // 2
    pltpu.sync_copy(src_hbm.at[pl.ds(wid * SEG_CPW, half), :], isrc)
    pltpu.sync_copy(dst_hbm.at[pl.ds(wid * SEG_CPW, half), :], idst)
    abase = jnp.minimum(sid * A_SLC, N_NODE - A_SLC)

    pltpu.sync_copy(zrows_hbm.at[pl.ds(0, SEG_SUB), :], rows[0])
    for k in range(A_SLC // SEG_SUB):
      pltpu.sync_copy(rows[0], acc.at[pl.ds(abase + k * SEG_SUB, SEG_SUB), :])

    plsc.subcore_barrier()

    def make_body(base):
      def body(q, carry):
        descs = []
        for t in range(SEG_DEPTH):
          j = q * SEG_DEPTH + t
          d = pltpu.make_async_copy(table_hbm.at[isrc.at[j]], rows[t],
                                    sems[t])
          descs.append((j, d))

          @pl.when(base + j < nch)
          def _(d=d):
            d.start()

        for t, (j, d) in enumerate(descs):
          @pl.when(base + j < nch)
          def _(t=t, j=j, d=d):
            d.wait()
            pltpu.sync_copy(rows[t], acc.at[idst.at[j]], add=True)

        return carry

      return body

    lax.fori_loop(0, SEG_GROUPS // 2, make_body(0), 0)
    pltpu.sync_copy(src_hbm.at[pl.ds(wid * SEG_CPW + half, half), :], isrc)
    pltpu.sync_copy(dst_hbm.at[pl.ds(wid * SEG_CPW + half, half), :], idst)
    lax.fori_loop(0, SEG_GROUPS // 2, make_body(half), 0)
    plsc.subcore_barrier()

    for k in range(A_SLC // SEG_SUB):
      pltpu.sync_copy(acc.at[pl.ds(abase + k * SEG_SUB, SEG_SUB), :], rows[0])
      pltpu.sync_copy(
          rows[0],
          s_hbm.at[pl.ds(cid * N_NODE + abase + k * SEG_SUB, SEG_SUB), :])

  return seg_sum


_seg_sum = _make_seg_sum(False)


BLK = 1000
_P = lax.Precision.HIGHEST


def _l2norm(o):
  n = jnp.sqrt(jnp.sum(o * o, axis=-1, keepdims=True))
  return o / jnp.maximum(n, 1e-12)


def _tc_new_p_body(xp, cpa, cpb, er, wl0rp, bl0rp, wr0rp, br0rp, out_p):
  cp = cpa[...] + cpb[...]
  rowp = jnp.dot(er[...], wl0rp[...], precision=_P,
                 preferred_element_type=jnp.float32)
  o_p = (jnp.where(cp > 0.0, 1.0, 0.0) * rowp
         + jnp.dot(xp[...], wr0rp[...], precision=_P,
                   preferred_element_type=jnp.float32)
         + bl0rp[...] + br0rp[...])
  out_p[...] = jax.nn.relu(_l2norm(o_p))


def _tc_new_r_body(s0a, s0b, cra, crb, er, wl0pr, bl0pr, wr0pr, br0pr,
                   out_r):
  c = cra[...] + crb[...]
  mean = (s0a[...] + s0b[...]) / jnp.maximum(c, 1.0)
  row0 = (jnp.dot(er[...], wr0pr[...], precision=_P,
                  preferred_element_type=jnp.float32)
          + bl0pr[...] + br0pr[...])
  o_r = jnp.dot(mean, wl0pr[...], precision=_P,
                preferred_element_type=jnp.float32) + row0
  out_r[...] = jax.nn.relu(_l2norm(o_r))


def _tc_layer1_body(s1a, s1b, cra, crb, nr,
                    wl1pr, bl1pr, wr1pr, br1pr, wout, bout, out):
  c = cra[...] + crb[...]
  mean = (s1a[...] + s1b[...]) / jnp.maximum(c, 1.0)
  o = (jnp.dot(mean, wl1pr[...], precision=_P,
               preferred_element_type=jnp.float32) + bl1pr[...]
       + jnp.dot(nr[...], wr1pr[...], precision=_P,
                 preferred_element_type=jnp.float32) + br1pr[...])
  t = jax.nn.relu(_l2norm(o))
  out[...] = jnp.dot(t, wout[...], precision=_P,
                     preferred_element_type=jnp.float32) + bout[...]


def _rows_spec(width=D):
  return pl.BlockSpec((BLK, width), lambda i: (i, 0))


def _full_spec(shape):
  return pl.BlockSpec(shape, lambda i: tuple(0 for _ in shape))


def kernel(x_protein, x_reaction, edge_index_p2r, edge_index_r2p,
           emb_protein, emb_reaction,
           Wl0pr, bl0pr, Wr0pr, br0pr, Wl0rp, bl0rp, Wr0rp, br0rp,
           Wl1pr, bl1pr, Wr1pr, br1pr, Wl1rp, bl1rp, Wr1rp, br1rp,
           W_out, b_out):
  f32 = jnp.float32
  xpid = x_protein[:, 0].astype(jnp.int32)

  def _chunked(ix):
    ix = ix.astype(jnp.int32).reshape(N_CHUNKS, CHUNK)
    return jnp.pad(ix, ((0, N_CHUNKS_PAD - N_CHUNKS), (0, 0)))

  src2 = _chunked(edge_index_p2r[0])
  dst2 = _chunked(edge_index_p2r[1])
  rdst2 = _chunked(edge_index_r2p[1])

  def _seg_chunked(ix):
    ix = ix.astype(jnp.int32).reshape(SEG_NCH, SEG_CHUNK)
    return jnp.pad(ix, ((0, SEG_PAD - SEG_NCH), (0, 0)))

  srcS = _seg_chunked(edge_index_p2r[0])
  dstS = _seg_chunked(edge_index_p2r[1])
  ones_c = jnp.ones((CHUNK,), f32)
  zrows = jnp.zeros((CHUNK, D), f32)
  zh = jnp.zeros((H_SLC,), f32)

  xp, cp_part, cr_part = _sc_nodes_and_cp(emb_protein.astype(f32), xpid,
                                          rdst2, dst2, ones_c, zh)
  (s0_part,) = _seg_sum(xp, srcS, dstS, ones_c, zrows, zh)

  er = emb_reaction.astype(f32).reshape(1, D)
  row = lambda b: b.reshape(1, D)
  cp_a = cp_part[:N_NODE].reshape(N_NODE, 1)
  cp_b = cp_part[N_NODE:].reshape(N_NODE, 1)
  cr_a = cr_part[:N_NODE].reshape(N_NODE, 1)
  cr_b = cr_part[N_NODE:].reshape(N_NODE, 1)
  s0_a, s0_b = s0_part[:N_NODE], s0_part[N_NODE:]

  grid = (N_NODE // BLK,)
  new_p = pl.pallas_call(
      _tc_new_p_body,
      grid=grid,
      in_specs=[
          _rows_spec(), _rows_spec(1), _rows_spec(1),
          _full_spec((1, D)),
          _full_spec((D, D)), _full_spec((1, D)),
          _full_spec((D, D)), _full_spec((1, D)),
      ],
      out_specs=_rows_spec(),
      out_shape=jax.ShapeDtypeStruct((N_NODE, D), f32),
  )(xp, cp_a, cp_b, er, Wl0rp, row(bl0rp), Wr0rp, row(br0rp))

  (s1_part,) = _seg_sum(new_p, srcS, dstS, ones_c, zrows, zh)
  s1_a, s1_b = s1_part[:N_NODE], s1_part[N_NODE:]

  new_r = pl.pallas_call(
      _tc_new_r_body,
      grid=grid,
      in_specs=[
          _rows_spec(), _rows_spec(), _rows_spec(1), _rows_spec(1),
          _full_spec((1, D)),
          _full_spec((D, D)), _full_spec((1, D)),
          _full_spec((D, D)), _full_spec((1, D)),
      ],
      out_specs=_rows_spec(),
      out_shape=jax.ShapeDtypeStruct((N_NODE, D), f32),
  )(s0_a, s0_b, cr_a, cr_b, er, Wl0pr, row(bl0pr), Wr0pr, row(br0pr))

  out = pl.pallas_call(
      _tc_layer1_body,
      grid=grid,
      in_specs=[
          _rows_spec(), _rows_spec(), _rows_spec(1), _rows_spec(1),
          _rows_spec(),
          _full_spec((D, D)), _full_spec((1, D)),
          _full_spec((D, D)), _full_spec((1, D)),
          _full_spec((D, D)), _full_spec((1, D)),
      ],
      out_specs=_rows_spec(),
      out_shape=jax.ShapeDtypeStruct((N_NODE, D), f32),
  )(s1_a, s1_b, cr_a, cr_b, new_r,
    Wl1pr, row(bl1pr), Wr1pr, row(br1pr), W_out, row(b_out))

  return out


# speedup vs baseline: 1.0250x; 1.0250x over previous
import functools

import jax
import jax.numpy as jnp
from jax import lax
from jax.experimental import pallas as pl
from jax.experimental.pallas import tpu as pltpu
from jax.experimental.pallas import tpu_sc as plsc

N_NODE = 10000
N_EDGE = 160000
D = 128

NC, NS = 2, 16
NW = NC * NS
CHUNK = 128
N_CHUNKS = N_EDGE // CHUNK
CPW = (N_CHUNKS + NW - 1) // NW
CPW_LAST = N_CHUNKS - (NW - 1) * CPW
N_CHUNKS_PAD = NW * CPW
DEPTH = 2
GROUPS = CPW // DEPTH

NG_FULL = N_NODE // CHUNK
NG_TAIL = N_NODE - NG_FULL * CHUNK
NG_ITERS = (NG_FULL + NW - 1) // NW

H_SLC = 640
A_SLC = 640
A_SUB = CHUNK

_mesh = plsc.VectorSubcoreMesh(core_axis_name="c", subcore_axis_name="s")


def _worker_id():
  cid = lax.axis_index("c")
  sid = lax.axis_index("s")
  return cid, sid, sid * NC + cid


@functools.partial(
    pl.kernel,
    out_type=(
        jax.ShapeDtypeStruct((N_NODE, D), jnp.float32),
        jax.ShapeDtypeStruct((NC * N_NODE,), jnp.float32),
        jax.ShapeDtypeStruct((NC * N_NODE,), jnp.float32),
    ),
    mesh=_mesh,
    scratch_types=(
        tuple(pltpu.VMEM((CHUNK,), jnp.int32) for _ in range(NG_ITERS + 1)),
        pltpu.VMEM((CPW, CHUNK), jnp.int32),
        pltpu.VMEM((CPW, CHUNK), jnp.int32),
        tuple(pltpu.VMEM((CHUNK, D), jnp.float32) for _ in range(NG_ITERS + 1)),
        pltpu.VMEM((CHUNK,), jnp.float32),
        pltpu.VMEM((H_SLC,), jnp.float32),
        pltpu.VMEM_SHARED((N_NODE,), jnp.float32),
        pltpu.VMEM_SHARED((N_NODE,), jnp.float32),
        tuple(pltpu.SemaphoreType.DMA for _ in range(NG_ITERS + 1)),
    ),
)
def _sc_nodes_and_cp(emb_hbm, xpid_hbm, rdst_hbm, pdst_hbm, ones_hbm, zh_hbm,
                     xp_hbm, cp_hbm, cr_hbm,
                     idxs, ridx, didx, rows, ones_v, hbuf, cp_acc, cr_acc,
                     sems):
  cid, sid, wid = _worker_id()
  nch = jnp.where(wid == NW - 1, CPW_LAST, CPW)
  pltpu.sync_copy(ones_hbm, ones_v)
  hbase = jnp.minimum(sid * H_SLC, N_NODE - H_SLC)

  for c in range(NG_ITERS):
    g = c * NW + wid

    @pl.when(g < NG_FULL)
    def _(c=c, g=g):
      pltpu.sync_copy(xpid_hbm.at[pl.ds(g * CHUNK, CHUNK)], idxs[c])
      pltpu.make_async_copy(emb_hbm.at[idxs[c]], rows[c], sems[c]).start()

  tail = pl.ds(0, NG_TAIL)

  @pl.when(wid == 0)
  def _():
    pltpu.sync_copy(xpid_hbm.at[pl.ds(NG_FULL * CHUNK, NG_TAIL)],
                    idxs[NG_ITERS].at[tail])
    pltpu.make_async_copy(emb_hbm.at[idxs[NG_ITERS].at[tail]],
                          rows[NG_ITERS].at[tail], sems[NG_ITERS]).start()

  pltpu.sync_copy(rdst_hbm.at[pl.ds(wid * CPW, CPW), :], ridx)
  pltpu.sync_copy(pdst_hbm.at[pl.ds(wid * CPW, CPW), :], didx)

  pltpu.sync_copy(zh_hbm, hbuf)
  pltpu.sync_copy(hbuf, cp_acc.at[pl.ds(hbase, H_SLC)])
  pltpu.sync_copy(hbuf, cr_acc.at[pl.ds(hbase, H_SLC)])

  for c in range(NG_ITERS):
    g = c * NW + wid

    @pl.when(g < NG_FULL)
    def _(c=c, g=g):
      pltpu.make_async_copy(emb_hbm.at[idxs[c]], rows[c], sems[c]).wait()
      pltpu.sync_copy(rows[c], xp_hbm.at[pl.ds(g * CHUNK, CHUNK), :])

  @pl.when(wid == 0)
  def _():
    pltpu.make_async_copy(emb_hbm.at[idxs[NG_ITERS].at[tail]],
                          rows[NG_ITERS].at[tail], sems[NG_ITERS]).wait()
    pltpu.sync_copy(rows[NG_ITERS].at[tail],
                    xp_hbm.at[pl.ds(NG_FULL * CHUNK, NG_TAIL), :])

  plsc.subcore_barrier()

  def body(q, carry):
    descs = []
    for t in range(2):
      j = q * 2 + t
      d0 = pltpu.make_async_copy(ones_v, cp_acc.at[ridx.at[j]], sems[2 * t])
      d1 = pltpu.make_async_copy(ones_v, cr_acc.at[didx.at[j]],
                                 sems[2 * t + 1])
      descs.append((j, d0))
      descs.append((j, d1))

      @pl.when(j < nch)
      def _(d0=d0, d1=d1):
        d0.start(add=True)
        d1.start(add=True)

    for (j, d) in descs:
      @pl.when(j < nch)
      def _(d=d):
        d.wait()

    return carry

  lax.fori_loop(0, CPW // 2, body, 0)
  plsc.subcore_barrier()

  pltpu.sync_copy(cp_acc.at[pl.ds(hbase, H_SLC)], hbuf)
  pltpu.sync_copy(hbuf, cp_hbm.at[pl.ds(cid * N_NODE + hbase, H_SLC)])
  pltpu.sync_copy(cr_acc.at[pl.ds(hbase, H_SLC)], hbuf)
  pltpu.sync_copy(hbuf, cr_hbm.at[pl.ds(cid * N_NODE + hbase, H_SLC)])


def _make_seg_sum(with_hist):
  out_type = [jax.ShapeDtypeStruct((NC * N_NODE, D), jnp.float32)]
  if with_hist:
    out_type.append(jax.ShapeDtypeStruct((NC * N_NODE,), jnp.float32))

  @functools.partial(
      pl.kernel,
      out_type=tuple(out_type),
      mesh=_mesh,
      scratch_types=(
          pltpu.VMEM((CPW, CHUNK), jnp.int32),
          pltpu.VMEM((CPW, CHUNK), jnp.int32),
          tuple(pltpu.VMEM((CHUNK, D), jnp.float32) for _ in range(DEPTH)),
          pltpu.VMEM_SHARED((N_NODE, D), jnp.float32),
          tuple(pltpu.SemaphoreType.DMA for _ in range(DEPTH)),
      ),
  )
  def seg_sum(table_hbm, src_hbm, dst_hbm, ones_hbm, zrows_hbm, zh_hbm,
              *rest):
    (s_hbm, isrc, idst, rows, acc, sems) = rest
    cid, sid, wid = _worker_id()
    nch = jnp.where(wid == NW - 1, CPW_LAST, CPW)
    pltpu.sync_copy(src_hbm.at[pl.ds(wid * CPW, CPW), :], isrc)
    pltpu.sync_copy(dst_hbm.at[pl.ds(wid * CPW, CPW), :], idst)
    abase = jnp.minimum(sid * A_SLC, N_NODE - A_SLC)

    pltpu.sync_copy(zrows_hbm, rows[0])
    for k in range(A_SLC // A_SUB):
      pltpu.sync_copy(rows[0], acc.at[pl.ds(abase + k * A_SUB, A_SUB), :])

    plsc.subcore_barrier()

    def body(q, carry):
      descs = []
      for t in range(DEPTH):
        j = q * DEPTH + t
        d = pltpu.make_async_copy(table_hbm.at[isrc.at[j]], rows[t], sems[t])
        descs.append((j, d))

        @pl.when(j < nch)
        def _(d=d):
          d.start()

      for t, (j, d) in enumerate(descs):
        @pl.when(j < nch)
        def _(t=t, j=j, d=d):
          d.wait()
          pltpu.sync_copy(rows[t], acc.at[idst.at[j]], add=True)

      return carry

    lax.fori_loop(0, GROUPS, body, 0)
    plsc.subcore_barrier()

    for k in range(A_SLC // A_SUB):
      pltpu.sync_copy(acc.at[pl.ds(abase + k * A_SUB, A_SUB), :], rows[0])
      pltpu.sync_copy(
          rows[0], s_hbm.at[pl.ds(cid * N_NODE + abase + k * A_SUB, A_SUB), :])

  return seg_sum


_seg_sum = _make_seg_sum(False)


BLK = 1000
_P = lax.Precision.HIGHEST


def _l2norm(o):
  n = jnp.sqrt(jnp.sum(o * o, axis=-1, keepdims=True))
  return o / jnp.maximum(n, 1e-12)


def _tc_new_p_body(xp, cpa, cpb, er, wl0rp, bl0rp, wr0rp, br0rp, out_p):
  cp = cpa[...] + cpb[...]
  rowp = jnp.dot(er[...], wl0rp[...], precision=_P,
                 preferred_element_type=jnp.float32)
  o_p = (jnp.where(cp > 0.0, 1.0, 0.0) * rowp
         + jnp.dot(xp[...], wr0rp[...], precision=_P,
                   preferred_element_type=jnp.float32)
         + bl0rp[...] + br0rp[...])
  out_p[...] = jax.nn.relu(_l2norm(o_p))


def _tc_new_r_body(s0a, s0b, cra, crb, er, wl0pr, bl0pr, wr0pr, br0pr,
                   out_r):
  c = cra[...] + crb[...]
  mean = (s0a[...] + s0b[...]) / jnp.maximum(c, 1.0)
  row0 = (jnp.dot(er[...], wr0pr[...], precision=_P,
                  preferred_element_type=jnp.float32)
          + bl0pr[...] + br0pr[...])
  o_r = jnp.dot(mean, wl0pr[...], precision=_P,
                preferred_element_type=jnp.float32) + row0
  out_r[...] = jax.nn.relu(_l2norm(o_r))


def _tc_layer1_body(s1a, s1b, cra, crb, nr,
                    wl1pr, bl1pr, wr1pr, br1pr, wout, bout, out):
  c = cra[...] + crb[...]
  mean = (s1a[...] + s1b[...]) / jnp.maximum(c, 1.0)
  o = (jnp.dot(mean, wl1pr[...], precision=_P,
               preferred_element_type=jnp.float32) + bl1pr[...]
       + jnp.dot(nr[...], wr1pr[...], precision=_P,
                 preferred_element_type=jnp.float32) + br1pr[...])
  t = jax.nn.relu(_l2norm(o))
  out[...] = jnp.dot(t, wout[...], precision=_P,
                     preferred_element_type=jnp.float32) + bout[...]


def _rows_spec(width=D):
  return pl.BlockSpec((BLK, width), lambda i: (i, 0))


def _full_spec(shape):
  return pl.BlockSpec(shape, lambda i: tuple(0 for _ in shape))


def kernel(x_protein, x_reaction, edge_index_p2r, edge_index_r2p,
           emb_protein, emb_reaction,
           Wl0pr, bl0pr, Wr0pr, br0pr, Wl0rp, bl0rp, Wr0rp, br0rp,
           Wl1pr, bl1pr, Wr1pr, br1pr, Wl1rp, bl1rp, Wr1rp, br1rp,
           W_out, b_out):
  f32 = jnp.float32
  xpid = x_protein[:, 0].astype(jnp.int32)

  def _chunked(ix):
    ix = ix.astype(jnp.int32).reshape(N_CHUNKS, CHUNK)
    return jnp.pad(ix, ((0, N_CHUNKS_PAD - N_CHUNKS), (0, 0)))

  src2 = _chunked(edge_index_p2r[0])
  dst2 = _chunked(edge_index_p2r[1])
  rdst2 = _chunked(edge_index_r2p[1])
  ones_c = jnp.ones((CHUNK,), f32)
  zrows = jnp.zeros((CHUNK, D), f32)
  zh = jnp.zeros((H_SLC,), f32)

  xp, cp_part, cr_part = _sc_nodes_and_cp(emb_protein.astype(f32), xpid,
                                          rdst2, dst2, ones_c, zh)
  (s0_part,) = _seg_sum(xp, src2, dst2, ones_c, zrows, zh)

  er = emb_reaction.astype(f32).reshape(1, D)
  row = lambda b: b.reshape(1, D)
  cp_a = cp_part[:N_NODE].reshape(N_NODE, 1)
  cp_b = cp_part[N_NODE:].reshape(N_NODE, 1)
  cr_a = cr_part[:N_NODE].reshape(N_NODE, 1)
  cr_b = cr_part[N_NODE:].reshape(N_NODE, 1)
  s0_a, s0_b = s0_part[:N_NODE], s0_part[N_NODE:]

  grid = (N_NODE // BLK,)
  new_p = pl.pallas_call(
      _tc_new_p_body,
      grid=grid,
      in_specs=[
          _rows_spec(), _rows_spec(1), _rows_spec(1),
          _full_spec((1, D)),
          _full_spec((D, D)), _full_spec((1, D)),
          _full_spec((D, D)), _full_spec((1, D)),
      ],
      out_specs=_rows_spec(),
      out_shape=jax.ShapeDtypeStruct((N_NODE, D), f32),
  )(xp, cp_a, cp_b, er, Wl0rp, row(bl0rp), Wr0rp, row(br0rp))

  (s1_part,) = _seg_sum(new_p, src2, dst2, ones_c, zrows, zh)
  s1_a, s1_b = s1_part[:N_NODE], s1_part[N_NODE:]

  new_r = pl.pallas_call(
      _tc_new_r_body,
      grid=grid,
      in_specs=[
          _rows_spec(), _rows_spec(), _rows_spec(1), _rows_spec(1),
          _full_spec((1, D)),
          _full_spec((D, D)), _full_spec((1, D)),
          _full_spec((D, D)), _full_spec((1, D)),
      ],
      out_specs=_rows_spec(),
      out_shape=jax.ShapeDtypeStruct((N_NODE, D), f32),
  )(s0_a, s0_b, cr_a, cr_b, er, Wl0pr, row(bl0pr), Wr0pr, row(br0pr))

  out = pl.pallas_call(
      _tc_layer1_body,
      grid=grid,
      in_specs=[
          _rows_spec(), _rows_spec(), _rows_spec(1), _rows_spec(1),
          _rows_spec(),
          _full_spec((D, D)), _full_spec((1, D)),
          _full_spec((D, D)), _full_spec((1, D)),
          _full_spec((D, D)), _full_spec((1, D)),
      ],
      out_specs=_rows_spec(),
      out_shape=jax.ShapeDtypeStruct((N_NODE, D), f32),
  )(s1_a, s1_b, cr_a, cr_b, new_r,
    Wl1pr, row(bl1pr), Wr1pr, row(br1pr), W_out, row(b_out))

  return out
